# baseline (speedup 1.0000x reference)
import functools
import math

import jax
import jax.numpy as jnp
from jax import lax
from jax.experimental import pallas as pl
from jax.experimental.pallas import tpu as pltpu
from jax.experimental.pallas import tpu_sc as plsc

B, N, D = 32, 8192, 768
H = D // 2
K = 256
TEMP = 0.1
BR = 1024
R = B * N

_NC = 2
_NS = 16
_NW = _NC * _NS
_CH = 64
_NCH = K // _CH
_L = 16
_NV = N // _L
_NL1 = _NV // _L
_GRP = _NV // _L


def _gelu_exact(x):
    f = jnp.float32
    z = (-x) * f(0.707106769)
    ax = jnp.abs(z)
    z2 = z * z
    pe = z2 * f(7.85386146e-05) + f(-0.000801019371)
    pe = pe * z2 + f(0.00518832775)
    pe = pe * z2 + f(-0.0268538129)
    pe = pe * z2 + f(0.112835854)
    pe = pe * z2 + f(-0.37612626)
    pe = pe * z2 + f(1.12837911)
    erfc_small = 1.0 - z * pe
    nz2 = -z2
    e = jnp.exp(nz2)
    q = 1.0 / ax
    ze = e * q
    y = 1.0 / z2
    ps = y * f(0.0232682) + f(-0.138703942)
    ps = ps * y + f(0.368742466)
    ps = ps * y + f(-0.582473278)
    ps = ps * y + f(0.621000469)
    ps = ps * y + f(-0.494451523)
    ps = ps * y + f(0.340488)
    ps = ps * y + f(-0.274112701)
    ps = ps * y + f(0.563825965)
    pl_ = y * f(-10.477664) + f(12.9772)
    pl_ = pl_ * y + f(-7.49551868)
    pl_ = pl_ * y + f(2.92101908)
    pl_ = pl_ * y + f(-1.01526523)
    pl_ = pl_ * y + f(0.42184633)
    pl_ = pl_ * y + f(-0.282076746)
    pl_ = pl_ * y + f(0.564189494)
    r = ze * jnp.where(ax < 2.0, ps, pl_)
    r = jnp.where(nz2 < f(-88.7228394), 0.0, r)
    r = jnp.where(z < 0.0, 2.0 - r, r)
    erfc = jnp.where(ax < 1.0, erfc_small, r)
    return (x * 0.5) * erfc


def _scores_body(a_ref, b_ref, w1_ref, b1_ref, w2_ref, b2_ref, out_ref):
    fused = (a_ref[...] + b_ref[...]) / 2.0
    h = None
    for o in (0, 256, 512):
        p = jnp.dot(fused[:, o:o + 256].astype(jnp.bfloat16),
                    w1_ref[o:o + 256, :].astype(jnp.bfloat16),
                    preferred_element_type=jnp.float32)
        h = p if h is None else h + p
    h = h + b1_ref[...]
    h = _gelu_exact(h)
    s = jnp.dot(h.astype(jnp.bfloat16), w2_ref[...].astype(jnp.bfloat16),
                preferred_element_type=jnp.float32)
    out_ref[...] = s + b2_ref[...]


def _scores(img2d, text2d, W1, b1, W2, b2):
    grid = (R // BR,)
    return pl.pallas_call(
        _scores_body,
        grid=grid,
        in_specs=[
            pl.BlockSpec((BR, D), lambda i: (i, 0)),
            pl.BlockSpec((BR, D), lambda i: (i, 0)),
            pl.BlockSpec((D, H), lambda i: (0, 0)),
            pl.BlockSpec((1, H), lambda i: (0, 0)),
            pl.BlockSpec((H, 1), lambda i: (0, 0)),
            pl.BlockSpec((1, 1), lambda i: (0, 0)),
        ],
        out_specs=pl.BlockSpec((BR, 1), lambda i: (i, 0)),
        out_shape=jax.ShapeDtypeStruct((R, 1), jnp.float32),
    )(img2d, text2d, W1, b1.reshape(1, H), W2, b2.reshape(1, 1))


def _softmax_body(s_ref, p_ref):
    x = s_ref[...] / TEMP
    m = jnp.max(x, axis=1, keepdims=True)
    e = jnp.exp(x - m)
    p_ref[...] = e / jnp.sum(e, axis=1, keepdims=True)


def _softmax(scores):
    return pl.pallas_call(
        _softmax_body,
        out_shape=jax.ShapeDtypeStruct((B, N), jnp.float32),
    )(scores)


def _sc_topk_gather(probs, img2d, text2d):
    mesh = plsc.VectorSubcoreMesh(core_axis_name="c", subcore_axis_name="s")

    @functools.partial(
        pl.kernel,
        mesh=mesh,
        out_type=(
            jax.ShapeDtypeStruct((B, K), jnp.int32),
            jax.ShapeDtypeStruct((B * K, D), jnp.float32),
        ),
        scratch_types=[
            pltpu.VMEM((N,), jnp.float32),
            pltpu.VMEM((N,), jnp.int32),
            pltpu.VMEM((_NV,), jnp.int32),
            pltpu.VMEM((2 * _L,), jnp.int32),
            pltpu.VMEM((K,), jnp.int32),
            pltpu.VMEM((K,), jnp.int32),
            pltpu.VMEM((_CH, D), jnp.float32),
            pltpu.VMEM((_CH, D), jnp.float32),
            pltpu.SemaphoreType.DMA,
            pltpu.SemaphoreType.DMA,
        ],
    )
    def topk_kernel(probs_hbm, img_hbm, text_hbm, idx_hbm, feats_hbm,
                    pv, wv, lvl1, lvl2, idxb, flatb, ga, gb,
                    sem_a, sem_b):
        wid = lax.axis_index("s") * _NC + lax.axis_index("c")
        liota = lax.broadcasted_iota(jnp.int32, (_L,), 0)
        pltpu.sync_copy(probs_hbm.at[wid], pv)

        def _chain_max(vec):
            m = vec[0]
            for t in range(1, _L):
                m = jnp.maximum(m, vec[t])
            return m

        def build_l1(j, c):
            lv = jnp.zeros((_L,), jnp.int32)
            for t in range(_L):
                d = lax.bitcast_convert_type(
                    pv[pl.ds((j * _L + t) * _L, _L)], jnp.int32)
                wv[pl.ds((j * _L + t) * _L, _L)] = d
                lv = jnp.where(liota == t, _chain_max(d), lv)
            lvl1[pl.ds(j * _L, _L)] = lv
            return c

        lax.fori_loop(0, _NL1, build_l1, 0)

        for h in range(2):
            lv = jnp.zeros((_L,), jnp.int32)
            for t in range(_L):
                lv = jnp.where(
                    liota == t,
                    _chain_max(lvl1[pl.ds((h * _L + t) * _L, _L)]), lv)
            lvl2[pl.ds(h * _L, _L)] = lv

        def outer(jo, c):
            def inner(ji, acc):
                idxvec, flatvec = acc
                l2a = lvl2[pl.ds(0, _L)]
                l2b = lvl2[pl.ds(_L, _L)]
                mv = l2a[0]
                j = jnp.int32(0)
                for t in range(1, _L):
                    cgt = l2a[t] > mv
                    mv = jnp.where(cgt, l2a[t], mv)
                    j = jnp.where(cgt, t, j)
                for t in range(_L):
                    cgt = l2b[t] > mv
                    mv = jnp.where(cgt, l2b[t], mv)
                    j = jnp.where(cgt, _L + t, j)
                l1v = lvl1[pl.ds(j * _L, _L)]
                vs = jnp.int32(0)
                for t in range(_L - 1, -1, -1):
                    vs = jnp.where(l1v[t] == mv, t, vs)
                v = j * _L + vs
                d = wv[pl.ds(v * _L, _L)]
                lane = jnp.int32(0)
                for t in range(_L - 1, -1, -1):
                    lane = jnp.where(d[t] == mv, t, lane)
                idx = v * _L + lane
                d2 = jnp.where(liota == lane, jnp.int32(-1), d)
                wv[pl.ds(v * _L, _L)] = d2
                l1n = jnp.where(liota == vs, _chain_max(d2), l1v)
                lvl1[pl.ds(j * _L, _L)] = l1n
                ng = _chain_max(l1n)
                lvl2[pl.ds(0, _L)] = jnp.where(liota == j, ng, l2a)
                lvl2[pl.ds(_L, _L)] = jnp.where(liota == j - _L, ng, l2b)
                idxvec = jnp.where(liota == ji, idx, idxvec)
                flatvec = jnp.where(liota == ji, idx + wid * N, flatvec)
                return (idxvec, flatvec)

            z = jnp.zeros((_L,), jnp.int32)
            idxvec, flatvec = lax.fori_loop(0, _L, inner, (z, z))
            idxb[pl.ds(jo * _L, _L)] = idxvec
            flatb[pl.ds(jo * _L, _L)] = flatvec
            return c

        lax.fori_loop(0, _L, outer, 0)
        pltpu.sync_copy(idxb, idx_hbm.at[wid])

        fbase = wid * K

        def chunk(ci, c):
            cidx = flatb.at[pl.ds(ci * _CH, _CH)]
            cp_a = pltpu.async_copy(img_hbm.at[cidx], ga, sem_a)
            cp_b = pltpu.async_copy(text_hbm.at[cidx], gb, sem_b)
            cp_a.wait()
            cp_b.wait()

            def row(r, c2):
                for j in range(D // _L):
                    sl = pl.ds(j * _L, _L)
                    ga[r, sl] = (ga[r, sl] + gb[r, sl]) / 2.0
                return c2

            lax.fori_loop(0, _CH, row, 0)
            pltpu.sync_copy(ga, feats_hbm.at[pl.ds(fbase + ci * _CH, _CH)])
            return c

        lax.fori_loop(0, _NCH, chunk, 0)

    return topk_kernel(probs, img2d, text2d)


def kernel(CLS_img, CLS_text, W1, b1, W2, b2, k):
    img2d = CLS_img.reshape(R, D)
    text2d = CLS_text.reshape(R, D)
    scores = _scores(img2d, text2d, W1, b1, W2, b2)
    probs = _softmax(scores.reshape(B, N))
    topk_indices, selected = _sc_topk_gather(probs, img2d, text2d)
    return (selected.reshape(B, K, D), topk_indices, probs)
